# baseline (speedup 1.0000x reference)
import functools

import jax
import jax.numpy as jnp
from jax import lax
from jax.experimental import pallas as pl
from jax.experimental.pallas import tpu as pltpu
from jax.experimental.pallas import tpu_sc as plsc

NC = 2
NS = 16
L = 16
NW = NC * NS

N = 4 * 8192
D = 1024
V = 8192
ROWS_PER_W = N // NW
C = 32
NCHUNK = ROWS_PER_W // C


def _body(x_hbm, idx_hbm, tab_hbm, out_hbm, idx_v, xa, gx, sem):
    wid = lax.axis_index("s") * NC + lax.axis_index("c")
    base = wid * ROWS_PER_W
    pltpu.sync_copy(idx_hbm.at[pl.ds(base, ROWS_PER_W)], idx_v)

    def chunk(c, carry):
        row0 = base + c * C
        pltpu.sync_copy(x_hbm.at[pl.ds(row0, C)], xa)
        pltpu.async_copy(tab_hbm.at[idx_v.at[pl.ds(c * C, C)]], gx, sem).wait()

        def row_add(r, carry2):
            for j in range(D // L):
                plsc.addupdate(xa.at[r, pl.ds(j * L, L)],
                               gx[r, pl.ds(j * L, L)])
            return carry2

        lax.fori_loop(0, C, row_add, 0)
        pltpu.sync_copy(xa, out_hbm.at[pl.ds(row0, C)])
        return carry

    lax.fori_loop(0, NCHUNK, chunk, 0)


@jax.jit
def _run(x2d, idx, tab):
    mesh = plsc.VectorSubcoreMesh(core_axis_name="c", subcore_axis_name="s")
    f = pl.kernel(
        _body,
        out_type=jax.ShapeDtypeStruct((N, D), jnp.float32),
        mesh=mesh,
        scratch_types=[
            pltpu.VMEM((ROWS_PER_W,), jnp.int32),
            pltpu.VMEM((C, D), jnp.float32),
            pltpu.VMEM((C, D), jnp.float32),
            pltpu.SemaphoreType.DMA,
        ],
    )
    return f(x2d, idx, tab)


def kernel(x, position_ids, pos_emb):
    B, Lseq, d = x.shape
    x2d = x.reshape(B * Lseq, d)
    idx = position_ids.reshape(-1).astype(jnp.int32)
    out = _run(x2d, idx, pos_emb)
    return out.reshape(B, Lseq, d)


# speedup vs baseline: 2.5501x; 2.5501x over previous
import functools

import jax
import jax.numpy as jnp
from jax import lax
from jax.experimental import pallas as pl
from jax.experimental.pallas import tpu as pltpu
from jax.experimental.pallas import tpu_sc as plsc

NC = 2
NS = 16
L = 16
NW = NC * NS

N = 4 * 8192
D = 1024
ROWS_PER_W = N // NW
C = 8
NCHUNK = ROWS_PER_W // C
NB = 4
LA = 2
NQUAD = NCHUNK // NB


def _body(x_hbm, idx_hbm, tab_hbm, out_hbm, idx_v, xa, gx, sx, sg, so):
    wid = lax.axis_index("s") * NC + lax.axis_index("c")
    base = wid * ROWS_PER_W
    pltpu.sync_copy(idx_hbm.at[pl.ds(base, ROWS_PER_W)], idx_v)

    def issue_loads(c, p):
        row0 = base + c * C
        pltpu.async_copy(x_hbm.at[pl.ds(row0, C)], xa.at[p], sx.at[p])
        pltpu.async_copy(tab_hbm.at[idx_v.at[pl.ds(c * C, C)]], gx.at[p],
                         sg.at[p])

    def wait_out(c, p):
        row0 = base + c * C
        pltpu.make_async_copy(xa.at[p], out_hbm.at[pl.ds(row0, C)],
                              so.at[p]).wait()

    def process(c, p):
        row0 = base + c * C
        pltpu.make_async_copy(x_hbm.at[pl.ds(row0, C)], xa.at[p],
                              sx.at[p]).wait()
        pltpu.make_async_copy(x_hbm.at[pl.ds(row0, C)], gx.at[p],
                              sg.at[p]).wait()

        def row_add(r, carry):
            for j in range(D // L):
                plsc.addupdate(xa.at[p, r, pl.ds(j * L, L)],
                               gx[p, r, pl.ds(j * L, L)])
            return carry

        lax.fori_loop(0, C, row_add, 0)
        pltpu.async_copy(xa.at[p], out_hbm.at[pl.ds(row0, C)], so.at[p])

    def step(c, p):
        cn = c + LA
        pn = (p + LA) % NB
        if isinstance(c, int):
            if cn < NCHUNK:
                if cn - NB >= 0:
                    wait_out(cn - NB, pn)
                issue_loads(cn, pn)
        else:
            wait_out(cn - NB, pn)
            issue_loads(cn, pn)
        process(c, p)

    for c in range(LA):
        issue_loads(c, c)
    for p in range(NB):
        step(p, p)

    def quad(g, carry):
        c0 = g * NB
        for p in range(NB):
            step(c0 + p, p)
        return carry

    lax.fori_loop(1, NQUAD - 1, quad, 0)

    for p in range(NB):
        step((NQUAD - 1) * NB + p, p)
    for p in range(NB):
        wait_out((NQUAD - 1) * NB + p, p)


@jax.jit
def _run(x2d, idx, tab):
    mesh = plsc.VectorSubcoreMesh(core_axis_name="c", subcore_axis_name="s")
    f = pl.kernel(
        _body,
        out_type=jax.ShapeDtypeStruct((N, D), jnp.float32),
        mesh=mesh,
        scratch_types=[
            pltpu.VMEM((ROWS_PER_W,), jnp.int32),
            pltpu.VMEM((NB, C, D), jnp.float32),
            pltpu.VMEM((NB, C, D), jnp.float32),
            pltpu.SemaphoreType.DMA((NB,)),
            pltpu.SemaphoreType.DMA((NB,)),
            pltpu.SemaphoreType.DMA((NB,)),
        ],
    )
    return f(x2d, idx, tab)


def kernel(x, position_ids, pos_emb):
    B, Lseq, d = x.shape
    x2d = x.reshape(B * Lseq, d)
    idx = position_ids.reshape(-1).astype(jnp.int32)
    out = _run(x2d, idx, pos_emb)
    return out.reshape(B, Lseq, d)
